# baseline (speedup 1.0000x reference)
import functools

import jax
import jax.numpy as jnp
from jax import lax
from jax.experimental import pallas as pl
from jax.experimental.pallas import tpu as pltpu
from jax.experimental.pallas import tpu_sc as plsc

VOCAB = 30522
HIDDEN = 768
MAX_POS = 512
BATCH = 64
SEQ = 512
EPS = 1e-12

NWORKERS = 32
NSEG = 4
SEGB = BATCH // NSEG
SEGTOK = SEGB * SEQ
TPW = SEGTOK // NWORKERS
CH = 32
NCHUNK = TPW // CH
NBUF = 4
OUTER = NCHUNK // NBUF


def _make_gather_kernel():
    mesh = plsc.VectorSubcoreMesh(core_axis_name="c", subcore_axis_name="s")

    @functools.partial(
        pl.kernel,
        out_type=jax.ShapeDtypeStruct((SEGTOK, HIDDEN), jnp.float32),
        mesh=mesh,
        scratch_types=[
            pltpu.VMEM((TPW,), jnp.int32),
            pltpu.VMEM((CH, HIDDEN), jnp.float32),
            pltpu.VMEM((CH, HIDDEN), jnp.float32),
            pltpu.VMEM((CH, HIDDEN), jnp.float32),
            pltpu.VMEM((CH, HIDDEN), jnp.float32),
            pltpu.SemaphoreType.DMA,
            pltpu.SemaphoreType.DMA,
            pltpu.SemaphoreType.DMA,
            pltpu.SemaphoreType.DMA,
            pltpu.SemaphoreType.DMA,
            pltpu.SemaphoreType.DMA,
            pltpu.SemaphoreType.DMA,
            pltpu.SemaphoreType.DMA,
        ],
    )
    def gather_kernel(ids_hbm, ttab_hbm, x_hbm,
                      ids_v, r0, r1, r2, r3,
                      g0, g1, g2, g3, o0, o1, o2, o3):
        rows = [r0, r1, r2, r3]
        gsems = [g0, g1, g2, g3]
        osems = [o0, o1, o2, o3]
        wid = lax.axis_index("s") * 2 + lax.axis_index("c")
        t0 = wid * TPW

        pltpu.sync_copy(ids_hbm.at[pl.ds(t0, TPW)], ids_v)

        def gather(ch, m):
            return pltpu.make_async_copy(
                ttab_hbm.at[ids_v.at[pl.ds(ch * CH, CH)]], rows[m], gsems[m])

        def outcopy(ch, m):
            return pltpu.make_async_copy(
                rows[m], x_hbm.at[pl.ds(t0 + ch * CH, CH), :], osems[m])

        gather(0, 0).start()
        gather(1, 1).start()

        def outer(g, carry):
            for k in range(NBUF):
                ch = g * NBUF + k
                m = k
                mp = (k + 2) % NBUF
                gather(ch, m).wait()
                outcopy(ch, m).start()

                @pl.when(ch + 2 < NCHUNK)
                def _():
                    @pl.when(ch >= 2)
                    def _():
                        outcopy(ch, mp).wait()
                    gather(ch + 2, mp).start()
            return carry

        lax.fori_loop(0, OUTER, outer, 0)
        for m in range(NBUF):
            outcopy(0, m).wait()

    return gather_kernel


_GATHER = _make_gather_kernel()


def _ln_body(x_ref, tid_ref, pos_ref, typ_ref, gam_ref, bet_ref, out_ref):
    x = x_ref[...]
    tidf = tid_ref[0].astype(jnp.float32)
    t0 = typ_ref[0:1, :]
    td = typ_ref[1:2, :] - t0
    xx = x + pos_ref[...] + (t0 + tidf * td)
    mean = jnp.mean(xx, axis=-1, keepdims=True)
    cen = xx - mean
    var = jnp.mean(cen * cen, axis=-1, keepdims=True)
    y = cen * lax.rsqrt(var + EPS)
    out_ref[...] = y * gam_ref[...] + bet_ref[...]


def _tc_ln(x, tids3, pos_table, type_table, gam2, bet2):
    return pl.pallas_call(
        _ln_body,
        grid=(SEGB,),
        in_specs=[
            pl.BlockSpec((SEQ, HIDDEN), lambda i: (i, 0)),
            pl.BlockSpec((1, SEQ, 1), lambda i: (i, 0, 0)),
            pl.BlockSpec((MAX_POS, HIDDEN), lambda i: (0, 0)),
            pl.BlockSpec((2, HIDDEN), lambda i: (0, 0)),
            pl.BlockSpec((1, HIDDEN), lambda i: (0, 0)),
            pl.BlockSpec((1, HIDDEN), lambda i: (0, 0)),
        ],
        out_specs=pl.BlockSpec((SEQ, HIDDEN), lambda i: (i, 0)),
        out_shape=jax.ShapeDtypeStruct((SEGTOK, HIDDEN), jnp.float32),
    )(x, tids3, pos_table, type_table, gam2, bet2)


def kernel(input_ids, token_type_ids, token_table, pos_table, type_table,
           ln_gamma, ln_beta):
    ids = input_ids.astype(jnp.int32).reshape(NSEG, SEGTOK)
    tids = token_type_ids.astype(jnp.int32).reshape(NSEG, SEGB, SEQ, 1)
    gam2 = ln_gamma.reshape(1, HIDDEN)
    bet2 = ln_beta.reshape(1, HIDDEN)
    xs = [_GATHER(ids[s], token_table) for s in range(NSEG)]

    def _xla_ln(x, tid3):
        xx = (x.reshape(SEGB, SEQ, HIDDEN) + pos_table[None]
              + type_table[tid3[..., 0]])
        mean = jnp.mean(xx, axis=-1, keepdims=True)
        cen = xx - mean
        var = jnp.mean(cen * cen, axis=-1, keepdims=True)
        y = cen * lax.rsqrt(var + EPS)
        return (y * gam2 + bet2).reshape(SEGTOK, HIDDEN)

    outs = [_xla_ln(xs[s], tids[s]) for s in range(NSEG)]
    return jnp.concatenate(outs, axis=0).reshape(BATCH, SEQ, HIDDEN)


# speedup vs baseline: 2.5019x; 2.5019x over previous
import functools

import jax
import jax.numpy as jnp
from jax import lax
from jax.experimental import pallas as pl
from jax.experimental.pallas import tpu as pltpu
from jax.experimental.pallas import tpu_sc as plsc

VOCAB = 30522
HIDDEN = 768
MAX_POS = 512
BATCH = 64
SEQ = 512
EPS = 1e-12

NWORKERS = 32
NSEG = 1
SEGB = BATCH // NSEG
SEGTOK = SEGB * SEQ
TPW = SEGTOK // NWORKERS
CH = 32
NCHUNK = TPW // CH
NBUF = 4
OUTER = NCHUNK // NBUF


def _make_gather_kernel():
    mesh = plsc.VectorSubcoreMesh(core_axis_name="c", subcore_axis_name="s")

    @functools.partial(
        pl.kernel,
        out_type=jax.ShapeDtypeStruct((SEGTOK, HIDDEN), jnp.float32),
        mesh=mesh,
        scratch_types=[
            pltpu.VMEM((TPW,), jnp.int32),
            pltpu.VMEM((CH, HIDDEN), jnp.float32),
            pltpu.VMEM((CH, HIDDEN), jnp.float32),
            pltpu.VMEM((CH, HIDDEN), jnp.float32),
            pltpu.VMEM((CH, HIDDEN), jnp.float32),
            pltpu.SemaphoreType.DMA,
            pltpu.SemaphoreType.DMA,
            pltpu.SemaphoreType.DMA,
            pltpu.SemaphoreType.DMA,
            pltpu.SemaphoreType.DMA,
            pltpu.SemaphoreType.DMA,
            pltpu.SemaphoreType.DMA,
            pltpu.SemaphoreType.DMA,
        ],
    )
    def gather_kernel(ids_hbm, ttab_hbm, x_hbm,
                      ids_v, r0, r1, r2, r3,
                      g0, g1, g2, g3, o0, o1, o2, o3):
        rows = [r0, r1, r2, r3]
        gsems = [g0, g1, g2, g3]
        osems = [o0, o1, o2, o3]
        wid = lax.axis_index("s") * 2 + lax.axis_index("c")
        t0 = wid * TPW

        pltpu.sync_copy(ids_hbm.at[pl.ds(t0, TPW)], ids_v)

        def gather(ch, m):
            return pltpu.make_async_copy(
                ttab_hbm.at[ids_v.at[pl.ds(ch * CH, CH)]], rows[m], gsems[m])

        def outcopy(ch, m):
            return pltpu.make_async_copy(
                rows[m], x_hbm.at[pl.ds(t0 + ch * CH, CH), :], osems[m])

        gather(0, 0).start()
        gather(1, 1).start()

        def outer(g, carry):
            for k in range(NBUF):
                ch = g * NBUF + k
                m = k
                mp = (k + 2) % NBUF
                gather(ch, m).wait()
                outcopy(ch, m).start()

                @pl.when(ch + 2 < NCHUNK)
                def _():
                    @pl.when(ch >= 2)
                    def _():
                        outcopy(ch, mp).wait()
                    gather(ch + 2, mp).start()
            return carry

        lax.fori_loop(0, OUTER, outer, 0)
        for m in range(NBUF):
            outcopy(0, m).wait()

    return gather_kernel


_GATHER = _make_gather_kernel()


def _ln_body(x_ref, tid_ref, pos_ref, typ_ref, gam_ref, bet_ref, out_ref):
    x = x_ref[...]
    tidf = tid_ref[0].astype(jnp.float32)
    t0 = typ_ref[0:1, :]
    td = typ_ref[1:2, :] - t0
    xx = x + pos_ref[...] + (t0 + tidf * td)
    mean = jnp.mean(xx, axis=-1, keepdims=True)
    cen = xx - mean
    var = jnp.mean(cen * cen, axis=-1, keepdims=True)
    y = cen * lax.rsqrt(var + EPS)
    out_ref[...] = y * gam_ref[...] + bet_ref[...]


def _tc_ln(x, tids3, pos_table, type_table, gam2, bet2):
    return pl.pallas_call(
        _ln_body,
        grid=(SEGB,),
        in_specs=[
            pl.BlockSpec((SEQ, HIDDEN), lambda i: (i, 0)),
            pl.BlockSpec((1, SEQ, 1), lambda i: (i, 0, 0)),
            pl.BlockSpec((MAX_POS, HIDDEN), lambda i: (0, 0)),
            pl.BlockSpec((2, HIDDEN), lambda i: (0, 0)),
            pl.BlockSpec((1, HIDDEN), lambda i: (0, 0)),
            pl.BlockSpec((1, HIDDEN), lambda i: (0, 0)),
        ],
        out_specs=pl.BlockSpec((SEQ, HIDDEN), lambda i: (i, 0)),
        out_shape=jax.ShapeDtypeStruct((SEGTOK, HIDDEN), jnp.float32),
    )(x, tids3, pos_table, type_table, gam2, bet2)


def kernel(input_ids, token_type_ids, token_table, pos_table, type_table,
           ln_gamma, ln_beta):
    ids = input_ids.astype(jnp.int32).reshape(NSEG, SEGTOK)
    tids = token_type_ids.astype(jnp.int32).reshape(NSEG, SEGB, SEQ, 1)
    gam2 = ln_gamma.reshape(1, HIDDEN)
    bet2 = ln_beta.reshape(1, HIDDEN)
    xs = [_GATHER(ids[s], token_table) for s in range(NSEG)]
    outs = [_tc_ln(xs[s], tids[s], pos_table, type_table, gam2, bet2)
            for s in range(NSEG)]
    return jnp.concatenate(outs, axis=0).reshape(BATCH, SEQ, HIDDEN)


# speedup vs baseline: 2.7188x; 1.0867x over previous
import functools

import jax
import jax.numpy as jnp
from jax import lax
from jax.experimental import pallas as pl
from jax.experimental.pallas import tpu as pltpu
from jax.experimental.pallas import tpu_sc as plsc

VOCAB = 30522
HIDDEN = 768
MAX_POS = 512
BATCH = 64
SEQ = 512
EPS = 1e-12

NWORKERS = 32
NSEG = 1
SEGB = BATCH // NSEG
SEGTOK = SEGB * SEQ
TPW = SEGTOK // NWORKERS
CH = 32
NCHUNK = TPW // CH
NBUF = 4
OUTER = NCHUNK // NBUF


def _make_gather_kernel():
    mesh = plsc.VectorSubcoreMesh(core_axis_name="c", subcore_axis_name="s")

    @functools.partial(
        pl.kernel,
        out_type=jax.ShapeDtypeStruct((SEGTOK, HIDDEN), jnp.float32),
        mesh=mesh,
        scratch_types=[
            pltpu.VMEM((TPW,), jnp.int32),
            pltpu.VMEM((CH, HIDDEN), jnp.float32),
            pltpu.VMEM((CH, HIDDEN), jnp.float32),
            pltpu.VMEM((CH, HIDDEN), jnp.float32),
            pltpu.VMEM((CH, HIDDEN), jnp.float32),
            pltpu.SemaphoreType.DMA,
            pltpu.SemaphoreType.DMA,
            pltpu.SemaphoreType.DMA,
            pltpu.SemaphoreType.DMA,
            pltpu.SemaphoreType.DMA,
            pltpu.SemaphoreType.DMA,
            pltpu.SemaphoreType.DMA,
            pltpu.SemaphoreType.DMA,
        ],
    )
    def gather_kernel(ids_hbm, ttab_hbm, x_hbm,
                      ids_v, r0, r1, r2, r3,
                      g0, g1, g2, g3, o0, o1, o2, o3):
        rows = [r0, r1, r2, r3]
        gsems = [g0, g1, g2, g3]
        osems = [o0, o1, o2, o3]
        wid = lax.axis_index("s") * 2 + lax.axis_index("c")
        t0 = wid * TPW

        pltpu.sync_copy(ids_hbm.at[pl.ds(t0, TPW)], ids_v)

        def gather(ch, m):
            return pltpu.make_async_copy(
                ttab_hbm.at[ids_v.at[pl.ds(ch * CH, CH)]], rows[m], gsems[m])

        def outcopy(ch, m):
            return pltpu.make_async_copy(
                rows[m], x_hbm.at[pl.ds(t0 + ch * CH, CH), :], osems[m])

        gather(0, 0).start()
        gather(1, 1).start()

        def outer(g, carry):
            for k in range(NBUF):
                ch = g * NBUF + k
                m = k
                mp = (k + 2) % NBUF
                gather(ch, m).wait()
                outcopy(ch, m).start()

                @pl.when(ch + 2 < NCHUNK)
                def _():
                    @pl.when(ch >= 2)
                    def _():
                        outcopy(ch, mp).wait()
                    gather(ch + 2, mp).start()
            return carry

        lax.fori_loop(0, OUTER, outer, 0)
        for m in range(NBUF):
            outcopy(0, m).wait()

    return gather_kernel


_GATHER = _make_gather_kernel()


BLKR = 2 * SEQ
NBLK = BATCH * SEQ // BLKR


def _ln_body(x_ref, tid_ref, pos_ref, typ_ref, gam_ref, bet_ref, out_ref):
    x = x_ref[...]
    tidf = tid_ref[0].astype(jnp.float32)
    t0 = typ_ref[0:1, :]
    td = typ_ref[1:2, :] - t0
    xx = x + pos_ref[...] + (t0 + tidf * td)
    mean = jnp.mean(xx, axis=-1, keepdims=True)
    cen = xx - mean
    var = jnp.mean(cen * cen, axis=-1, keepdims=True)
    y = cen * lax.rsqrt(var + EPS)
    out_ref[...] = y * gam_ref[...] + bet_ref[...]


def _tc_ln(x, tids3, pos_rep, type_table, gam2, bet2):
    return pl.pallas_call(
        _ln_body,
        grid=(NBLK,),
        in_specs=[
            pl.BlockSpec((BLKR, HIDDEN), lambda i: (i, 0)),
            pl.BlockSpec((1, BLKR, 1), lambda i: (i, 0, 0)),
            pl.BlockSpec((BLKR, HIDDEN), lambda i: (0, 0)),
            pl.BlockSpec((2, HIDDEN), lambda i: (0, 0)),
            pl.BlockSpec((1, HIDDEN), lambda i: (0, 0)),
            pl.BlockSpec((1, HIDDEN), lambda i: (0, 0)),
        ],
        out_specs=pl.BlockSpec((BLKR, HIDDEN), lambda i: (i, 0)),
        out_shape=jax.ShapeDtypeStruct((SEGTOK, HIDDEN), jnp.float32),
    )(x, tids3, pos_rep, type_table, gam2, bet2)


def kernel(input_ids, token_type_ids, token_table, pos_table, type_table,
           ln_gamma, ln_beta):
    ids = input_ids.astype(jnp.int32).reshape(SEGTOK)
    tids = token_type_ids.astype(jnp.int32).reshape(NBLK, BLKR, 1)
    gam2 = ln_gamma.reshape(1, HIDDEN)
    bet2 = ln_beta.reshape(1, HIDDEN)
    pos_rep = jnp.concatenate([pos_table, pos_table], axis=0)
    x = _GATHER(ids, token_table)
    out = _tc_ln(x, tids, pos_rep, type_table, gam2, bet2)
    return out.reshape(BATCH, SEQ, HIDDEN)


# speedup vs baseline: 2.7925x; 1.0271x over previous
import functools

import jax
import jax.numpy as jnp
from jax import lax
from jax.experimental import pallas as pl
from jax.experimental.pallas import tpu as pltpu
from jax.experimental.pallas import tpu_sc as plsc

VOCAB = 30522
HIDDEN = 768
MAX_POS = 512
BATCH = 64
SEQ = 512
EPS = 1e-12

NWORKERS = 32
NSEG = 1
SEGB = BATCH // NSEG
SEGTOK = SEGB * SEQ
TPW = SEGTOK // NWORKERS
CH = 32
NCHUNK = TPW // CH
NBUF = 4
OUTER = NCHUNK // NBUF


def _make_gather_kernel():
    mesh = plsc.VectorSubcoreMesh(core_axis_name="c", subcore_axis_name="s")

    @functools.partial(
        pl.kernel,
        out_type=jax.ShapeDtypeStruct((SEGTOK, HIDDEN), jnp.float32),
        mesh=mesh,
        scratch_types=[
            pltpu.VMEM((TPW,), jnp.int32),
            pltpu.VMEM((CH, HIDDEN), jnp.float32),
            pltpu.VMEM((CH, HIDDEN), jnp.float32),
            pltpu.VMEM((CH, HIDDEN), jnp.float32),
            pltpu.VMEM((CH, HIDDEN), jnp.float32),
            pltpu.SemaphoreType.DMA,
            pltpu.SemaphoreType.DMA,
            pltpu.SemaphoreType.DMA,
            pltpu.SemaphoreType.DMA,
            pltpu.SemaphoreType.DMA,
            pltpu.SemaphoreType.DMA,
            pltpu.SemaphoreType.DMA,
            pltpu.SemaphoreType.DMA,
        ],
    )
    def gather_kernel(ids_hbm, ttab_hbm, x_hbm,
                      ids_v, r0, r1, r2, r3,
                      g0, g1, g2, g3, o0, o1, o2, o3):
        rows = [r0, r1, r2, r3]
        gsems = [g0, g1, g2, g3]
        osems = [o0, o1, o2, o3]
        wid = lax.axis_index("s") * 2 + lax.axis_index("c")
        t0 = wid * TPW

        pltpu.sync_copy(ids_hbm.at[pl.ds(t0, TPW)], ids_v)

        def gather(ch, m):
            return pltpu.make_async_copy(
                ttab_hbm.at[ids_v.at[pl.ds(ch * CH, CH)]], rows[m], gsems[m])

        def outcopy(ch, m):
            return pltpu.make_async_copy(
                rows[m], x_hbm.at[pl.ds(t0 + ch * CH, CH), :], osems[m])

        gather(0, 0).start()
        gather(1, 1).start()

        def outer(g, carry):
            for k in range(NBUF):
                ch = g * NBUF + k
                m = k
                mp = (k + 2) % NBUF
                gather(ch, m).wait()
                outcopy(ch, m).start()

                @pl.when(ch + 2 < NCHUNK)
                def _():
                    @pl.when(ch >= 2)
                    def _():
                        outcopy(ch, mp).wait()
                    gather(ch + 2, mp).start()
            return carry

        lax.fori_loop(0, OUTER, outer, 0)
        for m in range(NBUF):
            outcopy(0, m).wait()

    return gather_kernel


_GATHER = _make_gather_kernel()


BLKR = 4 * SEQ
NBLK = BATCH * SEQ // BLKR


def _ln_body(x_ref, tid_ref, pos_ref, typ_ref, gam_ref, bet_ref, out_ref):
    x = x_ref[...]
    tidf = tid_ref[0].astype(jnp.float32)
    t0 = typ_ref[0:1, :]
    td = typ_ref[1:2, :] - t0
    xx = x + pos_ref[...] + (t0 + tidf * td)
    mean = jnp.mean(xx, axis=-1, keepdims=True)
    cen = xx - mean
    var = jnp.mean(cen * cen, axis=-1, keepdims=True)
    y = cen * lax.rsqrt(var + EPS)
    out_ref[...] = y * gam_ref[...] + bet_ref[...]


def _tc_ln(x, tids3, pos_rep, type_table, gam2, bet2):
    return pl.pallas_call(
        _ln_body,
        grid=(NBLK,),
        in_specs=[
            pl.BlockSpec((BLKR, HIDDEN), lambda i: (i, 0)),
            pl.BlockSpec((1, BLKR, 1), lambda i: (i, 0, 0)),
            pl.BlockSpec((BLKR, HIDDEN), lambda i: (0, 0)),
            pl.BlockSpec((2, HIDDEN), lambda i: (0, 0)),
            pl.BlockSpec((1, HIDDEN), lambda i: (0, 0)),
            pl.BlockSpec((1, HIDDEN), lambda i: (0, 0)),
        ],
        out_specs=pl.BlockSpec((BLKR, HIDDEN), lambda i: (i, 0)),
        out_shape=jax.ShapeDtypeStruct((SEGTOK, HIDDEN), jnp.float32),
    )(x, tids3, pos_rep, type_table, gam2, bet2)


def kernel(input_ids, token_type_ids, token_table, pos_table, type_table,
           ln_gamma, ln_beta):
    ids = input_ids.astype(jnp.int32).reshape(SEGTOK)
    tids = token_type_ids.astype(jnp.int32).reshape(NBLK, BLKR, 1)
    gam2 = ln_gamma.reshape(1, HIDDEN)
    bet2 = ln_beta.reshape(1, HIDDEN)
    pos_rep = jnp.concatenate([pos_table] * (BLKR // SEQ), axis=0)
    x = _GATHER(ids, token_table)
    out = _tc_ln(x, tids, pos_rep, type_table, gam2, bet2)
    return out.reshape(BATCH, SEQ, HIDDEN)
